# baseline (speedup 1.0000x reference)
import functools
import math

import jax
import jax.numpy as jnp
from jax.experimental import pallas as pl
from jax.experimental.pallas import tpu as pltpu


def _ctm_fused_kernel(x_ref, xm_ref, idx_ref, nw_ref, cid_ref, acc_ref,
                      *, k, S, C):
    N = x_ref.shape[1]
    for b in range(x_ref.shape[0]):
        _ctm_one_batch(x_ref, xm_ref, idx_ref, nw_ref, cid_ref, acc_ref, b,
                       k=k, S=S, C=C, N=N)


def _ctm_one_batch(x_ref, xm_ref, idx_ref, nw_ref, cid_ref, acc_ref, b,
                   *, k, S, C, N):
    xb = x_ref[b]
    sq = jnp.sum(xb * xb, axis=-1, keepdims=True)
    gram = jax.lax.dot_general(
        xb, xb, (((1,), (1,)), ((), ())),
        preferred_element_type=jnp.float32)
    d2 = jnp.maximum(sq + sq.T - 2.0 * gram, 0.0)

    cur = d2
    mins = []
    for it in range(k):
        m = jnp.min(cur, axis=-1, keepdims=True)
        mins.append(m)
        if it < k - 1:
            cur = jnp.where(cur == m, jnp.inf, cur)
    accf = mins[0]
    for m in mins[1:]:
        accf = accf + m
    acc_ref[...] = accf
    nbad = jnp.sum(jnp.where(cur == jnp.inf, 1.0, 0.0),
                   axis=-1, keepdims=True)
    dup_seen = jnp.max(nbad) != float(k - 1)

    @pl.when(dup_seen)
    def _exact_with_multiplicity():
        cur2 = d2
        acc = jnp.zeros((N, 1), jnp.float32)
        rem = jnp.full((N, 1), float(k), jnp.float32)
        for it in range(k):
            m = jnp.min(cur2, axis=-1, keepdims=True)
            if it == k - 1:
                acc = jnp.where(rem > 0, acc + m, acc)
                break
            hit = cur2 == m
            c = jnp.sum(jnp.where(hit, 1.0, 0.0), axis=-1, keepdims=True)
            t = jnp.minimum(c, rem)
            cand = acc
            for j in range(1, k + 1):
                cand = cand + m
                acc = jnp.where(t == float(j), cand, acc)
            rem = rem - t
            cur2 = jnp.where(hit, jnp.inf, cur2)
        acc_ref[...] = acc

    d2max = jnp.max(d2)
    acc = acc_ref[...]
    density = jnp.exp(-acc * (1.0 / (k * C)))
    density_row = density.T
    val = jnp.where(density_row > density, d2, d2max)
    dist2 = jnp.min(val, axis=-1, keepdims=True)
    dist = jnp.sqrt(dist2) * (1.0 / math.sqrt(C))
    score = dist * density
    score_row = score.T

    iota_col = jax.lax.broadcasted_iota(jnp.int32, (N, N), 1)
    iota_row = jax.lax.broadcasted_iota(jnp.int32, (N, N), 0)
    beats = (score_row > score) | ((score_row == score) & (iota_col < iota_row))
    rank = jnp.sum(jnp.where(beats, 1.0, 0.0), axis=-1,
                   keepdims=True).astype(jnp.int32)
    rank_row = rank.T

    n_i = jnp.int32(N)
    iota_s_cols = jax.lax.broadcasted_iota(jnp.int32, (N, S), 1)
    iota_n_rows = jax.lax.broadcasted_iota(jnp.int32, (N, S), 0)
    sel = rank == iota_s_cols
    index_down = jnp.min(jnp.where(sel, iota_n_rows, n_i),
                         axis=0, keepdims=True)
    cid_ref[b] = index_down

    iota_s_rows = jax.lax.broadcasted_iota(jnp.int32, (S, N), 0)
    sel_sn = rank_row == iota_s_rows
    sel_f = sel_sn.astype(jnp.float32)
    centers = jax.lax.dot_general(
        sel_f, xb, (((1,), (0,)), ((), ())),
        preferred_element_type=jnp.float32)
    sq_c = jnp.sum(jnp.where(sel_sn, sq.T, 0.0),
                   axis=-1, keepdims=True)
    gramc = jax.lax.dot_general(
        centers, xb, (((1,), (1,)), ((), ())),
        preferred_element_type=jnp.float32)
    d2c = jnp.maximum(sq_c + sq.T - 2.0 * gramc, 0.0)
    mmin = jnp.min(d2c, axis=0, keepdims=True)
    assign = jnp.min(jnp.where(d2c == mmin, iota_s_rows, jnp.int32(S)),
                     axis=0, keepdims=True)
    idx_final = jnp.where(rank_row < S, rank_row, assign)
    idx_ref[b] = idx_final

    onehot = (idx_final == iota_s_rows).astype(jnp.float32)
    counts = jnp.sum(onehot, axis=-1, keepdims=True)
    inv = 1.0 / (counts + 1e-6)
    xsum = jax.lax.dot_general(
        onehot, xb, (((1,), (0,)), ((), ())),
        preferred_element_type=jnp.float32)
    xm_ref[b] = xsum * inv
    nw_ref[b] = jnp.sum(onehot * inv, axis=0, keepdims=True)


def kernel(x, idx_token, agg_weight):
    B, N, C = x.shape
    S = max(math.ceil(N * 0.25), 1)
    k = 5 if 5 <= S else min(3, max(S // 2, 1))
    BB = 4 if B % 4 == 0 else (2 if B % 2 == 0 else 1)

    itemsize = jnp.dtype(x.dtype).itemsize
    cost = pl.CostEstimate(
        flops=B * (2 * N * N * C + 2 * S * N * C + (9 + 3 * k) * N * N),
        transcendentals=2 * B * N,
        bytes_accessed=B * (N * C * itemsize + S * C * 4 + 3 * N * 4 + S * 4),
    )
    xm, idxc, nw, cid = pl.pallas_call(
        functools.partial(_ctm_fused_kernel, k=k, S=S, C=C),
        out_shape=(
            jax.ShapeDtypeStruct((B, S, C), jnp.float32),
            jax.ShapeDtypeStruct((B, 1, N), jnp.int32),
            jax.ShapeDtypeStruct((B, 1, N), jnp.float32),
            jax.ShapeDtypeStruct((B, 1, S), jnp.int32),
        ),
        grid=(B // BB,),
        in_specs=[pl.BlockSpec((BB, N, C), lambda i: (i, 0, 0))],
        out_specs=(
            pl.BlockSpec((BB, S, C), lambda i: (i, 0, 0)),
            pl.BlockSpec((BB, 1, N), lambda i: (i, 0, 0)),
            pl.BlockSpec((BB, 1, N), lambda i: (i, 0, 0)),
            pl.BlockSpec((BB, 1, S), lambda i: (i, 0, 0)),
        ),
        scratch_shapes=[pltpu.VMEM((N, 1), jnp.float32)],
        compiler_params=pltpu.CompilerParams(
            dimension_semantics=("parallel",),
            vmem_limit_bytes=56 * 1024 * 1024),
        cost_estimate=cost,
    )(x)

    idx_cluster = idxc[:, 0, :]
    normw = nw[:, 0, :]
    idx_token_new = jnp.take_along_axis(idx_cluster, idx_token, axis=1)
    agg_weight_new = agg_weight * jnp.take_along_axis(
        normw, idx_token, axis=1)[..., None]
    return xm, idx_token_new, agg_weight_new, cid[:, 0, :]


# speedup vs baseline: 1.0021x; 1.0021x over previous
import functools
import math

import jax
import jax.numpy as jnp
from jax.experimental import pallas as pl
from jax.experimental.pallas import tpu as pltpu


def _ctm_fused_kernel(x_ref, xm_ref, idx_ref, nw_ref, cid_ref, acc_ref,
                      *, k, S, C):
    N = x_ref.shape[1]
    for b in range(x_ref.shape[0]):
        _ctm_one_batch(x_ref, xm_ref, idx_ref, nw_ref, cid_ref, acc_ref, b,
                       k=k, S=S, C=C, N=N)


def _ctm_one_batch(x_ref, xm_ref, idx_ref, nw_ref, cid_ref, acc_ref, b,
                   *, k, S, C, N):
    xb = x_ref[b]
    sq = jnp.sum(xb * xb, axis=-1, keepdims=True)
    gram = jax.lax.dot_general(
        xb, xb, (((1,), (1,)), ((), ())),
        preferred_element_type=jnp.float32)
    d2 = jnp.maximum(sq + sq.T - 2.0 * gram, 0.0)

    cur = d2
    mins = []
    for it in range(k):
        m = jnp.min(cur, axis=-1, keepdims=True)
        mins.append(m)
        if it < k - 1:
            cur = jnp.where(cur == m, jnp.inf, cur)
    accf = mins[0]
    for m in mins[1:]:
        accf = accf + m
    acc_ref[...] = accf
    nbad = jnp.sum(jnp.where(cur == jnp.inf, 1.0, 0.0),
                   axis=-1, keepdims=True)
    dup_seen = jnp.max(nbad) != float(k - 1)

    @pl.when(dup_seen)
    def _exact_with_multiplicity():
        cur2 = d2
        acc = jnp.zeros((N, 1), jnp.float32)
        rem = jnp.full((N, 1), float(k), jnp.float32)
        for it in range(k):
            m = jnp.min(cur2, axis=-1, keepdims=True)
            if it == k - 1:
                acc = jnp.where(rem > 0, acc + m, acc)
                break
            hit = cur2 == m
            c = jnp.sum(jnp.where(hit, 1.0, 0.0), axis=-1, keepdims=True)
            t = jnp.minimum(c, rem)
            cand = acc
            for j in range(1, k + 1):
                cand = cand + m
                acc = jnp.where(t == float(j), cand, acc)
            rem = rem - t
            cur2 = jnp.where(hit, jnp.inf, cur2)
        acc_ref[...] = acc

    d2max = jnp.max(d2)
    acc = acc_ref[...]
    density = jnp.exp(-acc * (1.0 / (k * C)))
    density_row = density.T
    val = jnp.where(density_row > density, d2, d2max)
    dist2 = jnp.min(val, axis=-1, keepdims=True)
    dist = jnp.sqrt(dist2) * (1.0 / math.sqrt(C))
    score = dist * density
    score_row = score.T

    iota_col = jax.lax.broadcasted_iota(jnp.int32, (N, N), 1)
    iota_row = jax.lax.broadcasted_iota(jnp.int32, (N, N), 0)
    beats = (score_row > score) | ((score_row == score) & (iota_col < iota_row))
    rank = jnp.sum(jnp.where(beats, 1.0, 0.0), axis=-1,
                   keepdims=True).astype(jnp.int32)
    rank_row = rank.T

    n_i = jnp.int32(N)
    iota_s_cols = jax.lax.broadcasted_iota(jnp.int32, (N, S), 1)
    iota_n_rows = jax.lax.broadcasted_iota(jnp.int32, (N, S), 0)
    sel = rank == iota_s_cols
    index_down = jnp.min(jnp.where(sel, iota_n_rows, n_i),
                         axis=0, keepdims=True)
    cid_ref[b] = index_down

    iota_s_rows = jax.lax.broadcasted_iota(jnp.int32, (S, N), 0)
    sel_sn = rank_row == iota_s_rows
    sel_f = sel_sn.astype(jnp.float32)
    centers = jax.lax.dot_general(
        sel_f, xb, (((1,), (0,)), ((), ())),
        preferred_element_type=jnp.float32)
    sq_c = jnp.sum(jnp.where(sel_sn, sq.T, 0.0),
                   axis=-1, keepdims=True)
    gramc = jax.lax.dot_general(
        centers, xb, (((1,), (1,)), ((), ())),
        preferred_element_type=jnp.float32)
    d2c = jnp.maximum(sq_c + sq.T - 2.0 * gramc, 0.0)
    mmin = jnp.min(d2c, axis=0, keepdims=True)
    assign = jnp.min(jnp.where(d2c == mmin, iota_s_rows, jnp.int32(S)),
                     axis=0, keepdims=True)
    idx_final = jnp.where(rank_row < S, rank_row, assign)
    idx_ref[b] = idx_final

    onehot = (idx_final == iota_s_rows).astype(jnp.float32)
    counts = jnp.sum(onehot, axis=-1, keepdims=True)
    inv = 1.0 / (counts + 1e-6)
    xsum = jax.lax.dot_general(
        onehot, xb, (((1,), (0,)), ((), ())),
        preferred_element_type=jnp.float32)
    xm_ref[b] = xsum * inv
    nw_ref[b] = jnp.sum(onehot * inv, axis=0, keepdims=True)


def kernel(x, idx_token, agg_weight):
    B, N, C = x.shape
    S = max(math.ceil(N * 0.25), 1)
    k = 5 if 5 <= S else min(3, max(S // 2, 1))
    BB = 2 if B % 2 == 0 else 1

    itemsize = jnp.dtype(x.dtype).itemsize
    cost = pl.CostEstimate(
        flops=B * (2 * N * N * C + 2 * S * N * C + (9 + 3 * k) * N * N),
        transcendentals=2 * B * N,
        bytes_accessed=B * (N * C * itemsize + S * C * 4 + 3 * N * 4 + S * 4),
    )
    xm, idxc, nw, cid = pl.pallas_call(
        functools.partial(_ctm_fused_kernel, k=k, S=S, C=C),
        out_shape=(
            jax.ShapeDtypeStruct((B, S, C), jnp.float32),
            jax.ShapeDtypeStruct((B, 1, N), jnp.int32),
            jax.ShapeDtypeStruct((B, 1, N), jnp.float32),
            jax.ShapeDtypeStruct((B, 1, S), jnp.int32),
        ),
        grid=(B // BB,),
        in_specs=[pl.BlockSpec((BB, N, C), lambda i: (i, 0, 0))],
        out_specs=(
            pl.BlockSpec((BB, S, C), lambda i: (i, 0, 0)),
            pl.BlockSpec((BB, 1, N), lambda i: (i, 0, 0)),
            pl.BlockSpec((BB, 1, N), lambda i: (i, 0, 0)),
            pl.BlockSpec((BB, 1, S), lambda i: (i, 0, 0)),
        ),
        scratch_shapes=[pltpu.VMEM((N, 1), jnp.float32)],
        compiler_params=pltpu.CompilerParams(
            dimension_semantics=("parallel",),
            vmem_limit_bytes=56 * 1024 * 1024),
        cost_estimate=cost,
    )(x)

    idx_cluster = idxc[:, 0, :]
    normw = nw[:, 0, :]
    idx_token_new = jnp.take_along_axis(idx_cluster, idx_token, axis=1)
    agg_weight_new = agg_weight * jnp.take_along_axis(
        normw, idx_token, axis=1)[..., None]
    return xm, idx_token_new, agg_weight_new, cid[:, 0, :]
